# Initial kernel scaffold; baseline (speedup 1.0000x reference)
import functools

import jax
import jax.numpy as jnp
from jax import lax
from jax.experimental import pallas as pl
from jax.experimental.pallas import tpu as pltpu
from jax.experimental.pallas import tpu_sc as plsc

B = 8
N = 20000
G = 50
L = 16
NT = 32
CHUNKS = B * N // L
BASE = CHUNKS // NT
EXTRA = CHUNKS - BASE * NT
CPB = N // L
EPS = 1e-7
IOU_THR = 0.1
LN2 = 0.6931471805599453
UNROLL = 5


def _logf(x):
    bits = lax.bitcast_convert_type(x, jnp.int32)
    e = lax.shift_right_logical(bits, 23) - 127
    m = lax.bitcast_convert_type((bits & 0x007FFFFF) | 0x3F800000, jnp.float32)
    big = m > 1.4142135381698608
    m = jnp.where(big, m * 0.5, m)
    e = e + big.astype(jnp.int32)
    t = (m - 1.0) / (m + 1.0)
    t2 = t * t
    p = 1.0 + t2 * (1 / 3 + t2 * (1 / 5 + t2 * (1 / 7 + t2 * (1 / 9))))
    return e.astype(jnp.float32) * LN2 + 2.0 * t * p


def _body(preds_hbm, gt_hbm, out_hbm, stage, gtraw,
          cnx, cny, cnw, cnh, cx2, cy2, car, accv):
    wid = lax.axis_index("s") * 2 + lax.axis_index("c")
    iota = lax.iota(jnp.int32, L)

    pltpu.sync_copy(preds_hbm.at[pl.ds(wid * (BASE * L * 5), BASE * L * 5)],
                    stage.at[pl.ds(0, BASE * L * 5)])

    @pl.when(wid < EXTRA)
    def _():
        pltpu.sync_copy(
            preds_hbm.at[pl.ds((BASE * NT + wid) * (L * 5), L * 5)],
            stage.at[pl.ds(BASE * L * 5, L * 5)])

    pltpu.sync_copy(gt_hbm, gtraw)

    for j in range(B * G // L):
        idx4 = (iota + j * L) * 4
        gx = plsc.load_gather(gtraw, [idx4])
        gy = plsc.load_gather(gtraw, [idx4 + 1])
        gw = plsc.load_gather(gtraw, [idx4 + 2])
        gh = plsc.load_gather(gtraw, [idx4 + 3])
        nx = gx / 512.0
        ny = gy / 512.0
        nw = gw / 512.0
        nh = gh / 512.0
        x2 = nx + nw
        y2 = ny + nh
        s = pl.ds(j * L, L)
        cnx[s] = nx
        cny[s] = ny
        cnw[s] = nw
        cnh[s] = nh
        cx2[s] = x2
        cy2[s] = y2
        car[s] = (x2 - nx) * (y2 - ny)

    zero = jnp.zeros((L,), jnp.float32)
    nchunks = jnp.where(wid < EXTRA, BASE + 1, BASE)

    def chunk_body(c, carry):
        bbox_a, conf_a, match_a = carry
        gc = jnp.where(c < BASE, wid * BASE + c, BASE * NT + wid)
        gtoff = (gc // CPB) * G
        pidx = iota * 5 + c * (L * 5)
        px = plsc.load_gather(stage, [pidx])
        py = plsc.load_gather(stage, [pidx + 1])
        pw = plsc.load_gather(stage, [pidx + 2])
        ph = plsc.load_gather(stage, [pidx + 3])
        pcf = plsc.load_gather(stage, [pidx + 4])
        ax2 = px + pw
        ay2 = py + ph
        area_a = (ax2 - px) * (ay2 - py)

        def g_body(i, gcarry):
            bi, bg = gcarry
            for k in range(UNROLL):
                gidx = jnp.full((L,), gtoff + (i * UNROLL + k), jnp.int32)
                bx1 = plsc.load_gather(cnx, [gidx])
                by1 = plsc.load_gather(cny, [gidx])
                bx2 = plsc.load_gather(cx2, [gidx])
                by2 = plsc.load_gather(cy2, [gidx])
                ab = plsc.load_gather(car, [gidx])
                ix = jnp.maximum(jnp.minimum(ax2, bx2) - jnp.maximum(px, bx1), 0.0)
                iy = jnp.maximum(jnp.minimum(ay2, by2) - jnp.maximum(py, by1), 0.0)
                inter = ix * iy
                union = area_a + ab - inter
                iou = inter / (union + EPS)
                upd = iou > bi
                bi = jnp.where(upd, iou, bi)
                bg = jnp.where(upd, gidx, bg)
            return bi, bg

        best_iou, best_g = lax.fori_loop(
            0, G // UNROLL, g_body,
            (zero - 1.0, jnp.zeros((L,), jnp.int32)))

        mf = (best_iou >= IOU_THR).astype(jnp.float32)
        tx = plsc.load_gather(cnx, [best_g])
        ty = plsc.load_gather(cny, [best_g])
        tw = plsc.load_gather(cnw, [best_g])
        th = plsc.load_gather(cnh, [best_g])
        s = zero
        for p, t in ((px, tx), (py, ty), (pw, tw), (ph, th)):
            d = p - t
            ad = jnp.abs(d)
            s = s + jnp.where(ad < 1.0, 0.5 * d * d, ad - 0.5)
        bbox_a = bbox_a + mf * s
        match_a = match_a + mf
        pcl = jnp.clip(pcf, EPS, 1.0 - EPS)
        conf_a = conf_a - (mf * _logf(pcl) + (1.0 - mf) * _logf(1.0 - pcl))
        return bbox_a, conf_a, match_a

    bbox_a, conf_a, match_a = lax.fori_loop(
        0, nchunks, chunk_body, (zero, zero, zero))

    accv[0] = bbox_a
    accv[1] = conf_a
    accv[2] = match_a
    pltpu.sync_copy(accv, out_hbm.at[wid])


_sc_call = functools.partial(
    pl.kernel,
    mesh=plsc.VectorSubcoreMesh(core_axis_name="c", subcore_axis_name="s"),
    out_type=jax.ShapeDtypeStruct((NT, 3, L), jnp.float32),
    scratch_types=[
        pltpu.VMEM(((BASE + 1) * L * 5,), jnp.float32),
        pltpu.VMEM((B * G * 4,), jnp.float32),
        pltpu.VMEM((B * G,), jnp.float32),
        pltpu.VMEM((B * G,), jnp.float32),
        pltpu.VMEM((B * G,), jnp.float32),
        pltpu.VMEM((B * G,), jnp.float32),
        pltpu.VMEM((B * G,), jnp.float32),
        pltpu.VMEM((B * G,), jnp.float32),
        pltpu.VMEM((B * G,), jnp.float32),
        pltpu.VMEM((3, L), jnp.float32),
    ],
)(_body)


def kernel(preds, images, gt_boxes):
    del images
    parts = _sc_call(preds.reshape(-1), gt_boxes.reshape(-1))
    sums = jnp.sum(parts, axis=(0, 2))
    tb, tc, tm = sums[0], sums[1], sums[2]
    total_bbox = jnp.where(tm > 0, tb / jnp.maximum(tm, 1.0), 0.0)
    total_conf = tc / (B * N)
    total_loss = total_bbox + total_conf
    return (total_loss, total_bbox, total_conf)


# speedup vs baseline: 3.9441x; 3.9441x over previous
import functools

import jax
import jax.numpy as jnp
from jax import lax
from jax.experimental import pallas as pl
from jax.experimental.pallas import tpu as pltpu
from jax.experimental.pallas import tpu_sc as plsc

B = 8
N = 20000
G = 50
L = 16
NT = 32
CHUNKS = B * N // L
BASE = CHUNKS // NT
EXTRA = CHUNKS - BASE * NT
CPB = N // L
EPS = 1e-7
IOU_THR = 0.1
LN2 = 0.6931471805599453
UNROLL = 5


def _logf(x):
    bits = lax.bitcast_convert_type(x, jnp.int32)
    e = lax.shift_right_logical(bits, 23) - 127
    m = lax.bitcast_convert_type((bits & 0x007FFFFF) | 0x3F800000, jnp.float32)
    big = m > 1.4142135381698608
    m = jnp.where(big, m * 0.5, m)
    e = e + big.astype(jnp.int32)
    t = (m - 1.0) / (m + 1.0)
    t2 = t * t
    p = 1.0 + t2 * (1 / 3 + t2 * (1 / 5 + t2 * (1 / 7 + t2 * (1 / 9))))
    return e.astype(jnp.float32) * LN2 + 2.0 * t * p


def _body(preds_hbm, gt_hbm, out_hbm, stage, gtraw,
          cnx, cny, cnw, cnh, cx2, cy2, car, accv):
    wid = lax.axis_index("s") * 2 + lax.axis_index("c")
    iota = lax.iota(jnp.int32, L)

    pltpu.sync_copy(preds_hbm.at[pl.ds(wid * (BASE * L * 5), BASE * L * 5)],
                    stage.at[pl.ds(0, BASE * L * 5)])

    @pl.when(wid < EXTRA)
    def _():
        pltpu.sync_copy(
            preds_hbm.at[pl.ds((BASE * NT + wid) * (L * 5), L * 5)],
            stage.at[pl.ds(BASE * L * 5, L * 5)])

    pltpu.sync_copy(gt_hbm, gtraw)

    for j in range(B * G // L):
        idx4 = (iota + j * L) * 4
        gx = plsc.load_gather(gtraw, [idx4])
        gy = plsc.load_gather(gtraw, [idx4 + 1])
        gw = plsc.load_gather(gtraw, [idx4 + 2])
        gh = plsc.load_gather(gtraw, [idx4 + 3])
        nx = gx / 512.0
        ny = gy / 512.0
        nw = gw / 512.0
        nh = gh / 512.0
        x2 = nx + nw
        y2 = ny + nh
        s = pl.ds(j * L, L)
        cnx[s] = nx
        cny[s] = ny
        cnw[s] = nw
        cnh[s] = nh
        cx2[s] = x2
        cy2[s] = y2
        car[s] = (x2 - nx) * (y2 - ny)

    zero = jnp.zeros((L,), jnp.float32)
    nchunks = jnp.where(wid < EXTRA, BASE + 1, BASE)

    def chunk_body(c, carry):
        bbox_a, conf_a, match_a = carry
        gc = jnp.where(c < BASE, wid * BASE + c, BASE * NT + wid)
        gtoff = (gc // CPB) * G
        pidx = iota * 5 + c * (L * 5)
        px = plsc.load_gather(stage, [pidx])
        py = plsc.load_gather(stage, [pidx + 1])
        pw = plsc.load_gather(stage, [pidx + 2])
        ph = plsc.load_gather(stage, [pidx + 3])
        pcf = plsc.load_gather(stage, [pidx + 4])
        ax2 = px + pw
        ay2 = py + ph
        area_a = (ax2 - px) * (ay2 - py)

        def g_body(i, gcarry):
            bi, bg = gcarry
            for k in range(UNROLL):
                gidx = jnp.full((L,), gtoff + (i * UNROLL + k), jnp.int32)
                bx1 = plsc.load_gather(cnx, [gidx])
                by1 = plsc.load_gather(cny, [gidx])
                bx2 = plsc.load_gather(cx2, [gidx])
                by2 = plsc.load_gather(cy2, [gidx])
                ab = plsc.load_gather(car, [gidx])
                ix = jnp.maximum(jnp.minimum(ax2, bx2) - jnp.maximum(px, bx1), 0.0)
                iy = jnp.maximum(jnp.minimum(ay2, by2) - jnp.maximum(py, by1), 0.0)
                inter = ix * iy
                union = area_a + ab - inter
                iou = inter / (union + EPS)
                upd = iou > bi
                bi = jnp.where(upd, iou, bi)
                bg = jnp.where(upd, gidx, bg)
            return bi, bg

        best_iou, best_g = lax.fori_loop(
            0, G // UNROLL, g_body,
            (zero - 1.0, jnp.zeros((L,), jnp.int32)))

        mf = (best_iou >= IOU_THR).astype(jnp.float32)
        tx = plsc.load_gather(cnx, [best_g])
        ty = plsc.load_gather(cny, [best_g])
        tw = plsc.load_gather(cnw, [best_g])
        th = plsc.load_gather(cnh, [best_g])
        s = zero
        for p, t in ((px, tx), (py, ty), (pw, tw), (ph, th)):
            d = p - t
            ad = jnp.abs(d)
            s = s + jnp.where(ad < 1.0, 0.5 * d * d, ad - 0.5)
        bbox_a = bbox_a + mf * s
        match_a = match_a + mf
        pcl = jnp.clip(pcf, EPS, 1.0 - EPS)
        conf_a = conf_a - (mf * _logf(pcl) + (1.0 - mf) * _logf(1.0 - pcl))
        return bbox_a, conf_a, match_a

    bbox_a, conf_a, match_a = lax.fori_loop(
        0, nchunks, chunk_body, (zero, zero, zero))

    accv[0] = bbox_a
    accv[1] = conf_a
    accv[2] = match_a
    pltpu.sync_copy(accv, out_hbm.at[wid])


_sc_call = functools.partial(
    pl.kernel,
    mesh=plsc.VectorSubcoreMesh(core_axis_name="c", subcore_axis_name="s"),
    out_type=jax.ShapeDtypeStruct((NT, 3, L), jnp.float32),
    compiler_params=pltpu.CompilerParams(needs_layout_passes=False),
    scratch_types=[
        pltpu.VMEM(((BASE + 1) * L * 5,), jnp.float32),
        pltpu.VMEM((B * G * 4,), jnp.float32),
        pltpu.VMEM((B * G,), jnp.float32),
        pltpu.VMEM((B * G,), jnp.float32),
        pltpu.VMEM((B * G,), jnp.float32),
        pltpu.VMEM((B * G,), jnp.float32),
        pltpu.VMEM((B * G,), jnp.float32),
        pltpu.VMEM((B * G,), jnp.float32),
        pltpu.VMEM((B * G,), jnp.float32),
        pltpu.VMEM((3, L), jnp.float32),
    ],
)(_body)


def kernel(preds, images, gt_boxes):
    del images
    parts = _sc_call(preds.reshape(-1), gt_boxes.reshape(-1))
    sums = jnp.sum(parts, axis=(0, 2))
    tb, tc, tm = sums[0], sums[1], sums[2]
    total_bbox = jnp.where(tm > 0, tb / jnp.maximum(tm, 1.0), 0.0)
    total_conf = tc / (B * N)
    total_loss = total_bbox + total_conf
    return (total_loss, total_bbox, total_conf)


# speedup vs baseline: 3.9678x; 1.0060x over previous
import functools

import jax
import jax.numpy as jnp
from jax import lax
from jax.experimental import pallas as pl
from jax.experimental.pallas import tpu as pltpu
from jax.experimental.pallas import tpu_sc as plsc

B = 8
N = 20000
G = 50
L = 16
NT = 32
CHUNKS = B * N // L
BASE = CHUNKS // NT
EXTRA = CHUNKS - BASE * NT
CPB = N // L
EPS = 1e-7
IOU_THR = 0.1
LN2 = 0.6931471805599453
UNROLL = 50


def _logf(x):
    bits = lax.bitcast_convert_type(x, jnp.int32)
    e = lax.shift_right_logical(bits, 23) - 127
    m = lax.bitcast_convert_type((bits & 0x007FFFFF) | 0x3F800000, jnp.float32)
    big = m > 1.4142135381698608
    m = jnp.where(big, m * 0.5, m)
    e = e + big.astype(jnp.int32)
    t = (m - 1.0) / (m + 1.0)
    t2 = t * t
    p = 1.0 + t2 * (1 / 3 + t2 * (1 / 5 + t2 * (1 / 7 + t2 * (1 / 9))))
    return e.astype(jnp.float32) * LN2 + 2.0 * t * p


def _body(preds_hbm, gt_hbm, out_hbm, stage, gtraw,
          cnx, cny, cnw, cnh, cx2, cy2, car, accv):
    wid = lax.axis_index("s") * 2 + lax.axis_index("c")
    iota = lax.iota(jnp.int32, L)

    pltpu.sync_copy(preds_hbm.at[pl.ds(wid * (BASE * L * 5), BASE * L * 5)],
                    stage.at[pl.ds(0, BASE * L * 5)])

    @pl.when(wid < EXTRA)
    def _():
        pltpu.sync_copy(
            preds_hbm.at[pl.ds((BASE * NT + wid) * (L * 5), L * 5)],
            stage.at[pl.ds(BASE * L * 5, L * 5)])

    pltpu.sync_copy(gt_hbm, gtraw)

    for j in range(B * G // L):
        idx4 = (iota + j * L) * 4
        gx = plsc.load_gather(gtraw, [idx4])
        gy = plsc.load_gather(gtraw, [idx4 + 1])
        gw = plsc.load_gather(gtraw, [idx4 + 2])
        gh = plsc.load_gather(gtraw, [idx4 + 3])
        nx = gx / 512.0
        ny = gy / 512.0
        nw = gw / 512.0
        nh = gh / 512.0
        x2 = nx + nw
        y2 = ny + nh
        s = pl.ds(j * L, L)
        cnx[s] = nx
        cny[s] = ny
        cnw[s] = nw
        cnh[s] = nh
        cx2[s] = x2
        cy2[s] = y2
        car[s] = (x2 - nx) * (y2 - ny)

    zero = jnp.zeros((L,), jnp.float32)
    nchunks = jnp.where(wid < EXTRA, BASE + 1, BASE)

    def chunk_body(c, carry):
        bbox_a, conf_a, match_a = carry
        gc = jnp.where(c < BASE, wid * BASE + c, BASE * NT + wid)
        gtoff = (gc // CPB) * G
        pidx = iota * 5 + c * (L * 5)
        px = plsc.load_gather(stage, [pidx])
        py = plsc.load_gather(stage, [pidx + 1])
        pw = plsc.load_gather(stage, [pidx + 2])
        ph = plsc.load_gather(stage, [pidx + 3])
        pcf = plsc.load_gather(stage, [pidx + 4])
        ax2 = px + pw
        ay2 = py + ph
        area_a = (ax2 - px) * (ay2 - py)

        def g_step(g, bi, bg):
            gidx = jnp.full((L,), gtoff + g, jnp.int32)
            bx1 = plsc.load_gather(cnx, [gidx])
            by1 = plsc.load_gather(cny, [gidx])
            bx2 = plsc.load_gather(cx2, [gidx])
            by2 = plsc.load_gather(cy2, [gidx])
            ab = plsc.load_gather(car, [gidx])
            ix = jnp.maximum(jnp.minimum(ax2, bx2) - jnp.maximum(px, bx1), 0.0)
            iy = jnp.maximum(jnp.minimum(ay2, by2) - jnp.maximum(py, by1), 0.0)
            inter = ix * iy
            union = area_a + ab - inter
            iou = inter / (union + EPS)
            upd = iou > bi
            bi = jnp.where(upd, iou, bi)
            bg = jnp.where(upd, gidx, bg)
            return bi, bg

        if UNROLL == G:
            best_iou = zero - 1.0
            best_g = jnp.zeros((L,), jnp.int32)
            for g in range(G):
                best_iou, best_g = g_step(g, best_iou, best_g)
        else:
            def g_body(i, gcarry):
                bi, bg = gcarry
                for k in range(UNROLL):
                    bi, bg = g_step(i * UNROLL + k, bi, bg)
                return bi, bg

            best_iou, best_g = lax.fori_loop(
                0, G // UNROLL, g_body,
                (zero - 1.0, jnp.zeros((L,), jnp.int32)))

        mf = (best_iou >= IOU_THR).astype(jnp.float32)
        tx = plsc.load_gather(cnx, [best_g])
        ty = plsc.load_gather(cny, [best_g])
        tw = plsc.load_gather(cnw, [best_g])
        th = plsc.load_gather(cnh, [best_g])
        s = zero
        for p, t in ((px, tx), (py, ty), (pw, tw), (ph, th)):
            d = p - t
            ad = jnp.abs(d)
            s = s + jnp.where(ad < 1.0, 0.5 * d * d, ad - 0.5)
        bbox_a = bbox_a + mf * s
        match_a = match_a + mf
        pcl = jnp.clip(pcf, EPS, 1.0 - EPS)
        conf_a = conf_a - (mf * _logf(pcl) + (1.0 - mf) * _logf(1.0 - pcl))
        return bbox_a, conf_a, match_a

    bbox_a, conf_a, match_a = lax.fori_loop(
        0, nchunks, chunk_body, (zero, zero, zero))

    accv[0] = bbox_a
    accv[1] = conf_a
    accv[2] = match_a
    pltpu.sync_copy(accv, out_hbm.at[wid])


_sc_call = functools.partial(
    pl.kernel,
    mesh=plsc.VectorSubcoreMesh(core_axis_name="c", subcore_axis_name="s"),
    out_type=jax.ShapeDtypeStruct((NT, 3, L), jnp.float32),
    compiler_params=pltpu.CompilerParams(needs_layout_passes=False),
    scratch_types=[
        pltpu.VMEM(((BASE + 1) * L * 5,), jnp.float32),
        pltpu.VMEM((B * G * 4,), jnp.float32),
        pltpu.VMEM((B * G,), jnp.float32),
        pltpu.VMEM((B * G,), jnp.float32),
        pltpu.VMEM((B * G,), jnp.float32),
        pltpu.VMEM((B * G,), jnp.float32),
        pltpu.VMEM((B * G,), jnp.float32),
        pltpu.VMEM((B * G,), jnp.float32),
        pltpu.VMEM((B * G,), jnp.float32),
        pltpu.VMEM((3, L), jnp.float32),
    ],
)(_body)


def kernel(preds, images, gt_boxes):
    del images
    parts = _sc_call(preds.reshape(-1), gt_boxes.reshape(-1))
    sums = jnp.sum(parts, axis=(0, 2))
    tb, tc, tm = sums[0], sums[1], sums[2]
    total_bbox = jnp.where(tm > 0, tb / jnp.maximum(tm, 1.0), 0.0)
    total_conf = tc / (B * N)
    total_loss = total_bbox + total_conf
    return (total_loss, total_bbox, total_conf)
